# baseline (speedup 1.0000x reference)
import functools

import jax
import jax.numpy as jnp
from jax import lax
from jax.experimental import pallas as pl
from jax.experimental.pallas import tpu as pltpu
from jax.experimental.pallas import tpu_sc as plsc

MAXI = 1024
B = 1024
O = 128
F = 256
NW = 32
O_PER_W = O // NW
FCHUNK = 8
NCHUNK = F // FCHUNK
ROWS = FCHUNK * O_PER_W
LANES = 16
NB0 = B // LANES


def _transpose_body(x_ref, i_ref, o_ref, it_ref):
    o_ref[...] = x_ref[...].T
    it_ref[...] = i_ref[...].T


def _transpose(x, i2):
    return pl.pallas_call(
        _transpose_body,
        out_shape=(
            jax.ShapeDtypeStruct((MAXI, MAXI), jnp.float32),
            jax.ShapeDtypeStruct((F, B), jnp.int32),
        ),
        grid=(1,),
        in_specs=[
            pl.BlockSpec((MAXI, MAXI), lambda i: (0, 0)),
            pl.BlockSpec((B, F), lambda i: (0, 0)),
        ],
        out_specs=(
            pl.BlockSpec((MAXI, MAXI), lambda i: (0, 0)),
            pl.BlockSpec((F, B), lambda i: (0, 0)),
        ),
    )(x, i2)


def _sc_lookup_body(lt_hbm, it_hbm, w_hbm, out_hbm,
                    widx_v, ib0, ib1, g0, g1, acc,
                    semib0, semib1, semg0, semg1):
    ibs = (ib0, ib1)
    gs = (g0, g1)
    semibs = (semib0, semib1)
    semgs = (semg0, semg1)

    wid = lax.axis_index("s") * 2 + lax.axis_index("c")

    pltpu.sync_copy(w_hbm.at[wid], widx_v)

    def start_chunk(c, p):
        pltpu.make_async_copy(
            it_hbm.at[pl.ds(c * FCHUNK, FCHUNK)], ibs[p], semibs[p]
        ).start()
        pltpu.make_async_copy(
            lt_hbm.at[widx_v.at[pl.ds(c * ROWS, ROWS)]], gs[p], semgs[p]
        ).start()

    def wait_chunk(p):
        pltpu.make_async_copy(
            it_hbm.at[pl.ds(0, FCHUNK)], ibs[p], semibs[p]
        ).wait()
        pltpu.make_async_copy(
            lt_hbm.at[widx_v.at[pl.ds(0, ROWS)]], gs[p], semgs[p]
        ).wait()

    zero = jnp.zeros((LANES,), jnp.float32)

    @plsc.parallel_loop(0, NB0, unroll=2)
    def zbody(i):
        b0 = i * LANES
        for lo in range(O_PER_W):
            acc[lo, pl.ds(b0, LANES)] = zero

    row_consts = [jnp.full((LANES,), r, jnp.int32) for r in range(ROWS)]

    def compute_chunk(p):
        ib = ibs[p]
        g = gs[p]

        @plsc.parallel_loop(0, NB0, unroll=2)
        def bbody(i):
            b0 = i * LANES
            ivecs = [ib[fl, pl.ds(b0, LANES)] for fl in range(FCHUNK)]
            for lo in range(O_PER_W):
                vals = [
                    plsc.load_gather(g.at[fl * O_PER_W + lo], [ivecs[fl]])
                    for fl in range(FCHUNK)
                ]
                while len(vals) > 1:
                    vals = [
                        vals[k] + vals[k + 1] if k + 1 < len(vals) else vals[k]
                        for k in range(0, len(vals), 2)
                    ]
                plsc.addupdate(acc.at[lo, pl.ds(b0, LANES)], vals[0])

    start_chunk(0, 0)
    start_chunk(1, 1)

    def ring_body(it, carry):
        c0 = it * 2
        for p in range(2):
            c = c0 + p
            wait_chunk(p)
            compute_chunk(p)
            start_chunk(c + 2, p)
        return carry

    lax.fori_loop(0, (NCHUNK - 2) // 2, ring_body, 0)

    for p in range(2):
        wait_chunk(p)
        compute_chunk(p)

    pltpu.sync_copy(acc, out_hbm.at[wid])


@functools.partial(
    pl.kernel,
    mesh=plsc.VectorSubcoreMesh(core_axis_name="c", subcore_axis_name="s"),
    compiler_params=pltpu.CompilerParams(
        needs_layout_passes=False, use_tc_tiling_on_sc=False
    ),
    out_type=jax.ShapeDtypeStruct((NW, O_PER_W, B), jnp.float32),
    scratch_types=[
        pltpu.VMEM((F * O_PER_W,), jnp.int32),
        pltpu.VMEM((FCHUNK, B), jnp.int32),
        pltpu.VMEM((FCHUNK, B), jnp.int32),
        pltpu.VMEM((ROWS, MAXI), jnp.float32),
        pltpu.VMEM((ROWS, MAXI), jnp.float32),
        pltpu.VMEM((O_PER_W, B), jnp.float32),
        pltpu.SemaphoreType.DMA,
        pltpu.SemaphoreType.DMA,
        pltpu.SemaphoreType.DMA,
        pltpu.SemaphoreType.DMA,
    ],
)
def _sc_lookup(lt_hbm, it_hbm, w_hbm, out_hbm, *rest):
    _sc_lookup_body(lt_hbm, it_hbm, w_hbm, out_hbm, *rest)


def kernel(input_indices, weight_indices, dense_fallback):
    input_flat = input_indices.reshape(B, -1)
    weight_flat = weight_indices.reshape(O, -1)

    wt = weight_flat.T.astype(jnp.int32)
    widx = wt.reshape(F, NW, O_PER_W).transpose(1, 0, 2).reshape(NW, F * O_PER_W)

    lt, it = _transpose(dense_fallback, input_flat.astype(jnp.int32))

    out3 = _sc_lookup(lt, it, widx)
    return out3.transpose(2, 0, 1).reshape(B, O)


# speedup vs baseline: 1.0446x; 1.0446x over previous
import functools

import jax
import jax.numpy as jnp
from jax import lax
from jax.experimental import pallas as pl
from jax.experimental.pallas import tpu as pltpu
from jax.experimental.pallas import tpu_sc as plsc

MAXI = 1024
B = 1024
O = 128
F = 256
NW = 32
O_PER_W = O // NW
FCHUNK = 8
NCHUNK = F // FCHUNK
ROWS = FCHUNK * O_PER_W
LANES = 16
NB0 = B // LANES


def _transpose_body(x_ref, o_ref):
    o_ref[...] = x_ref[...].T


def _transpose(x):
    return pl.pallas_call(
        _transpose_body,
        out_shape=jax.ShapeDtypeStruct((MAXI, MAXI), jnp.float32),
        grid=(1,),
        in_specs=[pl.BlockSpec((MAXI, MAXI), lambda i: (0, 0))],
        out_specs=pl.BlockSpec((MAXI, MAXI), lambda i: (0, 0)),
    )(x)


def _sc_lookup_body(lt_hbm, it_hbm, w_hbm, out_hbm,
                    widx_v, ib0, ib1, g0, g1, acc,
                    semib0, semib1, semg0, semg1):
    ibs = (ib0, ib1)
    gs = (g0, g1)
    semibs = (semib0, semib1)
    semgs = (semg0, semg1)

    wid = lax.axis_index("s") * 2 + lax.axis_index("c")

    pltpu.sync_copy(w_hbm.at[wid], widx_v)

    def start_chunk(c, p):
        pltpu.make_async_copy(
            it_hbm.at[pl.ds(c * FCHUNK, FCHUNK)], ibs[p], semibs[p]
        ).start()
        pltpu.make_async_copy(
            lt_hbm.at[widx_v.at[pl.ds(c * ROWS, ROWS)]], gs[p], semgs[p]
        ).start()

    def wait_chunk(p):
        pltpu.make_async_copy(
            it_hbm.at[pl.ds(0, FCHUNK)], ibs[p], semibs[p]
        ).wait()
        pltpu.make_async_copy(
            lt_hbm.at[widx_v.at[pl.ds(0, ROWS)]], gs[p], semgs[p]
        ).wait()

    zero = jnp.zeros((LANES,), jnp.float32)

    @plsc.parallel_loop(0, NB0, unroll=2)
    def zbody(i):
        b0 = i * LANES
        for lo in range(O_PER_W):
            acc[lo, pl.ds(b0, LANES)] = zero

    row_consts = [jnp.full((LANES,), r, jnp.int32) for r in range(ROWS)]

    def compute_chunk(p):
        ib = ibs[p]
        g = gs[p]

        @plsc.parallel_loop(0, NB0, unroll=2)
        def bbody(i):
            b0 = i * LANES
            ivecs = [ib[fl, pl.ds(b0, LANES)] for fl in range(FCHUNK)]
            for lo in range(O_PER_W):
                vals = [
                    plsc.load_gather(g.at[fl * O_PER_W + lo], [ivecs[fl]])
                    for fl in range(FCHUNK)
                ]
                while len(vals) > 1:
                    vals = [
                        vals[k] + vals[k + 1] if k + 1 < len(vals) else vals[k]
                        for k in range(0, len(vals), 2)
                    ]
                plsc.addupdate(acc.at[lo, pl.ds(b0, LANES)], vals[0])

    start_chunk(0, 0)
    start_chunk(1, 1)

    def ring_body(it, carry):
        c0 = it * 2
        for p in range(2):
            c = c0 + p
            wait_chunk(p)
            compute_chunk(p)
            start_chunk(c + 2, p)
        return carry

    lax.fori_loop(0, (NCHUNK - 2) // 2, ring_body, 0)

    for p in range(2):
        wait_chunk(p)
        compute_chunk(p)

    pltpu.sync_copy(acc, out_hbm.at[wid])


@functools.partial(
    pl.kernel,
    mesh=plsc.VectorSubcoreMesh(core_axis_name="c", subcore_axis_name="s"),
    compiler_params=pltpu.CompilerParams(
        needs_layout_passes=False, use_tc_tiling_on_sc=False
    ),
    out_type=jax.ShapeDtypeStruct((NW, O_PER_W, B), jnp.float32),
    scratch_types=[
        pltpu.VMEM((F * O_PER_W,), jnp.int32),
        pltpu.VMEM((FCHUNK, B), jnp.int32),
        pltpu.VMEM((FCHUNK, B), jnp.int32),
        pltpu.VMEM((ROWS, MAXI), jnp.float32),
        pltpu.VMEM((ROWS, MAXI), jnp.float32),
        pltpu.VMEM((O_PER_W, B), jnp.float32),
        pltpu.SemaphoreType.DMA,
        pltpu.SemaphoreType.DMA,
        pltpu.SemaphoreType.DMA,
        pltpu.SemaphoreType.DMA,
    ],
)
def _sc_lookup(lt_hbm, it_hbm, w_hbm, out_hbm, *rest):
    _sc_lookup_body(lt_hbm, it_hbm, w_hbm, out_hbm, *rest)


def kernel(input_indices, weight_indices, dense_fallback):
    input_flat = input_indices.reshape(B, -1)
    weight_flat = weight_indices.reshape(O, -1)

    it = input_flat.T.astype(jnp.int32)
    wt = weight_flat.T.astype(jnp.int32)
    widx = wt.reshape(F, NW, O_PER_W).transpose(1, 0, 2).reshape(NW, F * O_PER_W)

    lt = _transpose(dense_fallback)

    out3 = _sc_lookup(lt, it, widx)
    return out3.transpose(2, 0, 1).reshape(B, O)
